# baseline (speedup 1.0000x reference)
import functools

import jax
import jax.numpy as jnp
from jax import lax
from jax.experimental import pallas as pl
from jax.experimental.pallas import tpu as pltpu
from jax.experimental.pallas import tpu_sc as plsc

D = 1024
EPS = 1e-05
SEQ = 4096
C = 32
TC_BLOCK = 512
N_SPLIT = 2


@functools.lru_cache(maxsize=None)
def _make_gather_kernel(n_rows):
    info = plsc.get_sparse_core_info()
    nw = info.num_cores * info.num_subcores
    per_w = n_rows // nw
    n_g = per_w // C
    mesh = plsc.VectorSubcoreMesh(core_axis_name="c", subcore_axis_name="s")

    @functools.partial(
        pl.kernel,
        mesh=mesh,
        out_type=jax.ShapeDtypeStruct((n_rows, D), jnp.float32),
        compiler_params=pltpu.CompilerParams(needs_layout_passes=False),
        scratch_types=[
            pltpu.VMEM((per_w,), jnp.int32),
            pltpu.VMEM((C, D), jnp.float32),
            pltpu.VMEM((C, D), jnp.float32),
            pltpu.SemaphoreType.DMA,
            pltpu.SemaphoreType.DMA,
            pltpu.SemaphoreType.DMA,
            pltpu.SemaphoreType.DMA,
        ],
    )
    def k(ids_hbm, word_hbm, out_hbm, idx_v, buf0, buf1,
          semg0, semg1, semo0, semo1):
        bufs = (buf0, buf1)
        semg = (semg0, semg1)
        semo = (semo0, semo1)

        wid = lax.axis_index("s") * info.num_cores + lax.axis_index("c")
        base = wid * per_w
        pltpu.sync_copy(ids_hbm.at[pl.ds(base, per_w)], idx_v)

        def gather(g, p):
            row0 = pl.multiple_of(g * C, C)
            return pltpu.make_async_copy(
                word_hbm.at[idx_v.at[pl.ds(row0, C)]], bufs[p], semg[p])

        def writeout(g, p):
            row0 = pl.multiple_of(base + g * C, C)
            return pltpu.make_async_copy(
                bufs[p], out_hbm.at[pl.ds(row0, C)], semo[p])

        gather(0, 0).start()
        gather(1, 1).start()

        def step(g, p):
            gather(g, p).wait()
            writeout(g, p).start()
            writeout(g, p).wait()

            @pl.when(g + 2 < n_g)
            def _():
                gather(g + 2, p).start()

        def outer(go, carry):
            step(2 * go, 0)
            step(2 * go + 1, 1)
            return carry

        lax.fori_loop(0, n_g // 2, outer, 0)

    return k


def _ln_body(g_ref, p_ref, t_ref, w_ref, b_ref, o_ref):
    x = g_ref[...] + p_ref[...] + t_ref[...]
    mean = jnp.mean(x, axis=-1, keepdims=True)
    cx = x - mean
    var = jnp.mean(cx * cx, axis=-1, keepdims=True)
    y = cx * lax.rsqrt(var + EPS)
    o_ref[...] = y * w_ref[...] + b_ref[...]


@functools.lru_cache(maxsize=None)
def _make_ln_kernel(n_rows):
    n_blocks = n_rows // TC_BLOCK
    pos_blocks = SEQ // TC_BLOCK
    return pl.pallas_call(
        _ln_body,
        grid=(n_blocks,),
        in_specs=[
            pl.BlockSpec((TC_BLOCK, D), lambda i: (i, 0)),
            pl.BlockSpec((TC_BLOCK, D), lambda i: (i % pos_blocks, 0)),
            pl.BlockSpec((1, D), lambda i: (0, 0)),
            pl.BlockSpec((1, D), lambda i: (0, 0)),
            pl.BlockSpec((1, D), lambda i: (0, 0)),
        ],
        out_specs=pl.BlockSpec((TC_BLOCK, D), lambda i: (i, 0)),
        out_shape=jax.ShapeDtypeStruct((n_rows, D), jnp.float32),
    )


def kernel(input_ids, word_table, pos_table, type_table, ln_weight, ln_bias):
    b, s = input_ids.shape
    n = b * s
    nh = n // N_SPLIT
    ids_flat = jnp.reshape(input_ids.astype(jnp.int32), (n,))
    pos = pos_table[:SEQ]
    type_2d = jnp.reshape(type_table, (1, D))
    w_2d = jnp.reshape(ln_weight, (1, D))
    b_2d = jnp.reshape(ln_bias, (1, D))
    gk = _make_gather_kernel(nh)
    lk = _make_ln_kernel(nh)
    outs = []
    for h in range(N_SPLIT):
        gathered = gk(ids_flat[h * nh:(h + 1) * nh], word_table)
        outs.append(lk(gathered, pos, type_2d, w_2d, b_2d))
    return jnp.reshape(jnp.concatenate(outs, axis=0), (b, s, D))


# speedup vs baseline: 1.3272x; 1.3272x over previous
import functools

import jax
import jax.numpy as jnp
from jax import lax
from jax.experimental import pallas as pl
from jax.experimental.pallas import tpu as pltpu
from jax.experimental.pallas import tpu_sc as plsc

D = 1024
EPS = 1e-05
SEQ = 4096
C = 32
TC_BLOCK = 512


@functools.lru_cache(maxsize=None)
def _make_gather_kernel(n_rows):
    info = plsc.get_sparse_core_info()
    nw = info.num_cores * info.num_subcores
    per_w = n_rows // nw
    n_g = per_w // C
    mesh = plsc.VectorSubcoreMesh(core_axis_name="c", subcore_axis_name="s")

    @functools.partial(
        pl.kernel,
        mesh=mesh,
        out_type=jax.ShapeDtypeStruct((n_rows, D), jnp.float32),
        compiler_params=pltpu.CompilerParams(needs_layout_passes=False),
        scratch_types=[
            pltpu.VMEM((per_w,), jnp.int32),
            pltpu.VMEM((C, D), jnp.float32),
            pltpu.VMEM((C, D), jnp.float32),
            pltpu.SemaphoreType.DMA,
            pltpu.SemaphoreType.DMA,
            pltpu.SemaphoreType.DMA,
            pltpu.SemaphoreType.DMA,
        ],
    )
    def k(ids_hbm, word_hbm, out_hbm, idx_v, buf0, buf1,
          semg0, semg1, semo0, semo1):
        bufs = (buf0, buf1)
        semg = (semg0, semg1)
        semo = (semo0, semo1)

        wid = lax.axis_index("s") * info.num_cores + lax.axis_index("c")
        base = wid * per_w
        pltpu.sync_copy(ids_hbm.at[pl.ds(base, per_w)], idx_v)

        def gather(g, p):
            row0 = pl.multiple_of(g * C, C)
            return pltpu.make_async_copy(
                word_hbm.at[idx_v.at[pl.ds(row0, C)]], bufs[p], semg[p])

        def writeout(g, p):
            row0 = pl.multiple_of(base + g * C, C)
            return pltpu.make_async_copy(
                bufs[p], out_hbm.at[pl.ds(row0, C)], semo[p])

        gather(0, 0).start()
        gather(1, 1).start()

        def step(g, p):
            gather(g, p).wait()
            writeout(g, p).start()
            writeout(g, p).wait()

            @pl.when(g + 2 < n_g)
            def _():
                gather(g + 2, p).start()

        def outer(go, carry):
            step(2 * go, 0)
            step(2 * go + 1, 1)
            return carry

        lax.fori_loop(0, n_g // 2, outer, 0)

    return k


def _ln_body(g_ref, p_ref, t_ref, w_ref, b_ref, o_ref):
    x = g_ref[...] + p_ref[...] + t_ref[...]
    mean = jnp.mean(x, axis=-1, keepdims=True)
    cx = x - mean
    var = jnp.mean(cx * cx, axis=-1, keepdims=True)
    y = cx * lax.rsqrt(var + EPS)
    o_ref[...] = y * w_ref[...] + b_ref[...]


@functools.lru_cache(maxsize=None)
def _make_ln_kernel(n_rows):
    n_blocks = n_rows // TC_BLOCK
    pos_blocks = SEQ // TC_BLOCK
    return pl.pallas_call(
        _ln_body,
        grid=(n_blocks,),
        in_specs=[
            pl.BlockSpec((TC_BLOCK, D), lambda i: (i, 0)),
            pl.BlockSpec((TC_BLOCK, D), lambda i: (i % pos_blocks, 0)),
            pl.BlockSpec((1, D), lambda i: (0, 0)),
            pl.BlockSpec((1, D), lambda i: (0, 0)),
            pl.BlockSpec((1, D), lambda i: (0, 0)),
        ],
        out_specs=pl.BlockSpec((TC_BLOCK, D), lambda i: (i, 0)),
        out_shape=jax.ShapeDtypeStruct((n_rows, D), jnp.float32),
    )


def kernel(input_ids, word_table, pos_table, type_table, ln_weight, ln_bias):
    b, s = input_ids.shape
    n = b * s
    ids_flat = jnp.reshape(input_ids.astype(jnp.int32), (n,))
    gathered = _make_gather_kernel(n)(ids_flat, word_table)
    out = _make_ln_kernel(n)(
        gathered, pos_table[:SEQ], jnp.reshape(type_table, (1, D)),
        jnp.reshape(ln_weight, (1, D)), jnp.reshape(ln_bias, (1, D)))
    return jnp.reshape(out, (b, s, D))


# speedup vs baseline: 1.3554x; 1.0212x over previous
import functools

import jax
import jax.numpy as jnp
from jax import lax
from jax.experimental import pallas as pl
from jax.experimental.pallas import tpu as pltpu
from jax.experimental.pallas import tpu_sc as plsc

D = 1024
EPS = 1e-05
SEQ = 4096
C = 32
TC_BLOCK = 1024


@functools.lru_cache(maxsize=None)
def _make_gather_kernel(n_rows):
    info = plsc.get_sparse_core_info()
    nw = info.num_cores * info.num_subcores
    per_w = n_rows // nw
    n_g = per_w // C
    mesh = plsc.VectorSubcoreMesh(core_axis_name="c", subcore_axis_name="s")

    @functools.partial(
        pl.kernel,
        mesh=mesh,
        out_type=jax.ShapeDtypeStruct((n_rows, D), jnp.float32),
        compiler_params=pltpu.CompilerParams(needs_layout_passes=False),
        scratch_types=[
            pltpu.VMEM((per_w,), jnp.int32),
            pltpu.VMEM((C, D), jnp.float32),
            pltpu.VMEM((C, D), jnp.float32),
            pltpu.SemaphoreType.DMA,
            pltpu.SemaphoreType.DMA,
            pltpu.SemaphoreType.DMA,
            pltpu.SemaphoreType.DMA,
        ],
    )
    def k(ids_hbm, word_hbm, out_hbm, idx_v, buf0, buf1,
          semg0, semg1, semo0, semo1):
        bufs = (buf0, buf1)
        semg = (semg0, semg1)
        semo = (semo0, semo1)

        wid = lax.axis_index("s") * info.num_cores + lax.axis_index("c")
        base = wid * per_w
        pltpu.sync_copy(ids_hbm.at[pl.ds(base, per_w)], idx_v)

        def gather(g, p):
            row0 = pl.multiple_of(g * C, C)
            return pltpu.make_async_copy(
                word_hbm.at[idx_v.at[pl.ds(row0, C)]], bufs[p], semg[p])

        def writeout(g, p):
            row0 = pl.multiple_of(base + g * C, C)
            return pltpu.make_async_copy(
                bufs[p], out_hbm.at[pl.ds(row0, C)], semo[p])

        gather(0, 0).start()
        gather(1, 1).start()

        def step(g, p):
            gather(g, p).wait()
            writeout(g, p).start()
            writeout(g, p).wait()

            @pl.when(g + 2 < n_g)
            def _():
                gather(g + 2, p).start()

        def outer(go, carry):
            step(2 * go, 0)
            step(2 * go + 1, 1)
            return carry

        lax.fori_loop(0, n_g // 2, outer, 0)

    return k


def _ln_body(g_ref, p_ref, t_ref, w_ref, b_ref, o_ref):
    x = g_ref[...] + p_ref[...] + t_ref[...]
    mean = jnp.mean(x, axis=-1, keepdims=True)
    cx = x - mean
    var = jnp.mean(cx * cx, axis=-1, keepdims=True)
    y = cx * lax.rsqrt(var + EPS)
    o_ref[...] = y * w_ref[...] + b_ref[...]


@functools.lru_cache(maxsize=None)
def _make_ln_kernel(n_rows):
    n_blocks = n_rows // TC_BLOCK
    pos_blocks = SEQ // TC_BLOCK
    return pl.pallas_call(
        _ln_body,
        grid=(n_blocks,),
        in_specs=[
            pl.BlockSpec((TC_BLOCK, D), lambda i: (i, 0)),
            pl.BlockSpec((TC_BLOCK, D), lambda i: (i % pos_blocks, 0)),
            pl.BlockSpec((1, D), lambda i: (0, 0)),
            pl.BlockSpec((1, D), lambda i: (0, 0)),
            pl.BlockSpec((1, D), lambda i: (0, 0)),
        ],
        out_specs=pl.BlockSpec((TC_BLOCK, D), lambda i: (i, 0)),
        out_shape=jax.ShapeDtypeStruct((n_rows, D), jnp.float32),
    )


def kernel(input_ids, word_table, pos_table, type_table, ln_weight, ln_bias):
    b, s = input_ids.shape
    n = b * s
    ids_flat = jnp.reshape(input_ids.astype(jnp.int32), (n,))
    gathered = _make_gather_kernel(n)(ids_flat, word_table)
    out = _make_ln_kernel(n)(
        gathered, pos_table[:SEQ], jnp.reshape(type_table, (1, D)),
        jnp.reshape(ln_weight, (1, D)), jnp.reshape(ln_bias, (1, D)))
    return jnp.reshape(out, (b, s, D))


# speedup vs baseline: 1.3567x; 1.0010x over previous
import functools

import jax
import jax.numpy as jnp
from jax import lax
from jax.experimental import pallas as pl
from jax.experimental.pallas import tpu as pltpu
from jax.experimental.pallas import tpu_sc as plsc

D = 1024
EPS = 1e-05
SEQ = 4096
C = 32
TC_BLOCK = 2048


@functools.lru_cache(maxsize=None)
def _make_gather_kernel(n_rows):
    info = plsc.get_sparse_core_info()
    nw = info.num_cores * info.num_subcores
    per_w = n_rows // nw
    n_g = per_w // C
    mesh = plsc.VectorSubcoreMesh(core_axis_name="c", subcore_axis_name="s")

    @functools.partial(
        pl.kernel,
        mesh=mesh,
        out_type=jax.ShapeDtypeStruct((n_rows, D), jnp.float32),
        compiler_params=pltpu.CompilerParams(needs_layout_passes=False),
        scratch_types=[
            pltpu.VMEM((per_w,), jnp.int32),
            pltpu.VMEM((C, D), jnp.float32),
            pltpu.VMEM((C, D), jnp.float32),
            pltpu.SemaphoreType.DMA,
            pltpu.SemaphoreType.DMA,
            pltpu.SemaphoreType.DMA,
            pltpu.SemaphoreType.DMA,
        ],
    )
    def k(ids_hbm, word_hbm, out_hbm, idx_v, buf0, buf1,
          semg0, semg1, semo0, semo1):
        bufs = (buf0, buf1)
        semg = (semg0, semg1)
        semo = (semo0, semo1)

        wid = lax.axis_index("s") * info.num_cores + lax.axis_index("c")
        base = wid * per_w
        pltpu.sync_copy(ids_hbm.at[pl.ds(base, per_w)], idx_v)

        def gather(g, p):
            row0 = pl.multiple_of(g * C, C)
            return pltpu.make_async_copy(
                word_hbm.at[idx_v.at[pl.ds(row0, C)]], bufs[p], semg[p])

        def writeout(g, p):
            row0 = pl.multiple_of(base + g * C, C)
            return pltpu.make_async_copy(
                bufs[p], out_hbm.at[pl.ds(row0, C)], semo[p])

        gather(0, 0).start()
        gather(1, 1).start()

        def step(g, p):
            gather(g, p).wait()
            writeout(g, p).start()
            writeout(g, p).wait()

            @pl.when(g + 2 < n_g)
            def _():
                gather(g + 2, p).start()

        def outer(go, carry):
            step(2 * go, 0)
            step(2 * go + 1, 1)
            return carry

        lax.fori_loop(0, n_g // 2, outer, 0)

    return k


def _ln_body(g_ref, p_ref, t_ref, w_ref, b_ref, o_ref):
    x = g_ref[...] + p_ref[...] + t_ref[...]
    mean = jnp.mean(x, axis=-1, keepdims=True)
    cx = x - mean
    var = jnp.mean(cx * cx, axis=-1, keepdims=True)
    y = cx * lax.rsqrt(var + EPS)
    o_ref[...] = y * w_ref[...] + b_ref[...]


@functools.lru_cache(maxsize=None)
def _make_ln_kernel(n_rows):
    n_blocks = n_rows // TC_BLOCK
    pos_blocks = SEQ // TC_BLOCK
    return pl.pallas_call(
        _ln_body,
        grid=(n_blocks,),
        in_specs=[
            pl.BlockSpec((TC_BLOCK, D), lambda i: (i, 0)),
            pl.BlockSpec((TC_BLOCK, D), lambda i: (i % pos_blocks, 0)),
            pl.BlockSpec((1, D), lambda i: (0, 0)),
            pl.BlockSpec((1, D), lambda i: (0, 0)),
            pl.BlockSpec((1, D), lambda i: (0, 0)),
        ],
        out_specs=pl.BlockSpec((TC_BLOCK, D), lambda i: (i, 0)),
        out_shape=jax.ShapeDtypeStruct((n_rows, D), jnp.float32),
    )


def kernel(input_ids, word_table, pos_table, type_table, ln_weight, ln_bias):
    b, s = input_ids.shape
    n = b * s
    ids_flat = jnp.reshape(input_ids.astype(jnp.int32), (n,))
    gathered = _make_gather_kernel(n)(ids_flat, word_table)
    out = _make_ln_kernel(n)(
        gathered, pos_table[:SEQ], jnp.reshape(type_table, (1, D)),
        jnp.reshape(ln_weight, (1, D)), jnp.reshape(ln_bias, (1, D)))
    return jnp.reshape(out, (b, s, D))
